# baseline (speedup 1.0000x reference)
import functools

import jax
import jax.numpy as jnp
from jax import lax
from jax.experimental import pallas as pl
from jax.experimental.pallas import tpu as pltpu
from jax.experimental.pallas import tpu_sc as plsc

N = 5000
TOPK = 50
IOU_THRESH = 0.5

L = 16
NTILES = 16
NPAD = 5120
CHUNK = NPAD // NTILES
NVREG = CHUNK // L
LAST = N - (NTILES - 1) * CHUNK
NEG = float("-inf")
BIG = 1 << 30


def _nms_body(bxh, sch, outh, bb, scl, pub, loc, outv, sh, sem):
    cid = lax.axis_index("c")
    tid = lax.axis_index("s")

    @pl.when(cid == 0)
    def _():
        base = tid * CHUNK
        lanes = jnp.arange(L, dtype=jnp.int32)

        cp = pltpu.async_copy(bxh, bb.at[pl.ds(0, 4 * N)], sem)

        @pl.when(tid < NTILES - 1)
        def _():
            pltpu.sync_copy(sch.at[pl.ds(base, CHUNK)], scl)

        @pl.when(tid == NTILES - 1)
        def _():
            pltpu.sync_copy(sch.at[pl.ds((NTILES - 1) * CHUNK, LAST)],
                            scl.at[pl.ds(0, LAST)])
            for j in range(LAST // L, NVREG):
                off = j * L
                cur = scl[pl.ds(off, L)]
                mask = (lanes + off) < LAST
                scl[pl.ds(off, L)] = jnp.where(mask, cur,
                                               jnp.full((L,), NEG, jnp.float32))

        @pl.when(tid == 0)
        def _():
            zeros = jnp.zeros((L,), jnp.float32)
            for r in range(64):
                outv[pl.ds(r * L, L)] = zeros

        def publish(mloc, iloc, slot):
            pub[...] = jnp.where(
                lanes == 0, jnp.full((L,), mloc, jnp.float32),
                jnp.where(lanes == 1,
                          jnp.full((L,), iloc.astype(jnp.float32), jnp.float32),
                          jnp.zeros((L,), jnp.float32)))
            pltpu.sync_copy(pub.at[pl.ds(0, 8)],
                            sh.at[pl.ds(slot * (NTILES * 8) + tid * 8, 8)])

        def lane_reduce(bv, bi):
            mloc = jnp.max(bv, axis=0)
            iloc = jnp.min(jnp.where(bv == mloc, bi, BIG), axis=0)
            return mloc, iloc

        def amax(j, carry):
            bv, bi = carry
            v = scl[pl.ds(j * L, L)]
            ci = lanes + (base + j * L)
            upd = v > bv
            return (jnp.where(upd, v, bv), jnp.where(upd, ci, bi))

        bv0 = jnp.full((L,), NEG, jnp.float32)
        bi0 = lanes + base
        bv, bi = lax.fori_loop(0, NVREG, amax, (bv0, bi0), unroll=5)
        mloc, iloc = lane_reduce(bv, bi)
        publish(mloc, iloc, 0)
        cp.wait()
        plsc.subcore_barrier()

        def step(s, _):
            slot = lax.rem(s, 2)
            nslot = lax.rem(s + 1, 2)

            pltpu.sync_copy(sh.at[pl.ds(slot * (NTILES * 8), NTILES * 8)], loc)
            vals = plsc.load_gather(loc, [lanes * 8])
            idxs = plsc.load_gather(loc, [lanes * 8 + 1]).astype(jnp.int32)
            m = jnp.max(vals, axis=0)
            wi = jnp.min(jnp.where(vals == m, idxs, BIG), axis=0)
            valid = m > NEG

            w4 = jnp.full((L,), wi * 4, jnp.int32)
            x1w = plsc.load_gather(bb, [w4])
            y1w = plsc.load_gather(bb, [w4 + 1])
            x2w = plsc.load_gather(bb, [w4 + 2])
            y2w = plsc.load_gather(bb, [w4 + 3])
            aw = (x2w - x1w) * (y2w - y1w)

            def fuse(j, carry):
                bv, bi = carry
                off = base + j * L
                ci = lanes + off
                c4 = ci * 4
                x1 = plsc.load_gather(bb, [c4])
                y1 = plsc.load_gather(bb, [c4 + 1])
                x2 = plsc.load_gather(bb, [c4 + 2])
                y2 = plsc.load_gather(bb, [c4 + 3])
                a = (x2 - x1) * (y2 - y1)
                w = jnp.maximum(jnp.minimum(x2w, x2) - jnp.maximum(x1w, x1),
                                0.0)
                h = jnp.maximum(jnp.minimum(y2w, y2) - jnp.maximum(y1w, y1),
                                0.0)
                inter = w * h
                iou = inter / (aw + a - inter + jnp.float32(1e-8))
                kill = (iou >= IOU_THRESH) | (ci == wi)
                cur = scl[pl.ds(j * L, L)]
                newv = jnp.where(kill, NEG, cur)
                scl[pl.ds(j * L, L)] = newv
                upd = newv > bv
                return (jnp.where(upd, newv, bv), jnp.where(upd, ci, bi))

            bv, bi = lax.fori_loop(0, NVREG, fuse, (bv0, bi0), unroll=5)
            mloc, iloc = lane_reduce(bv, bi)
            publish(mloc, iloc, nslot)

            @pl.when(valid & (tid == 0))
            def _():
                msplat = jnp.full((L,), m, jnp.float32)
                zero = jnp.zeros((L,), jnp.float32)
                row = jnp.where(
                    lanes == 0, x1w,
                    jnp.where(lanes == 1, y1w,
                              jnp.where(lanes == 2, x2w,
                                        jnp.where(lanes == 3, y2w,
                                                  jnp.where(lanes == 4, msplat,
                                                            zero)))))
                outv[pl.ds(s * L, L)] = row

            plsc.subcore_barrier()
            return 0

        lax.fori_loop(0, TOPK, step, 0)

        @pl.when(tid == 0)
        def _():
            pltpu.sync_copy(outv, outh)


_nms_call = pl.kernel(
    _nms_body,
    out_type=jax.ShapeDtypeStruct((64 * L,), jnp.float32),
    mesh=plsc.VectorSubcoreMesh(core_axis_name="c", subcore_axis_name="s"),
    compiler_params=pltpu.CompilerParams(needs_layout_passes=False),
    scratch_types=[
        pltpu.VMEM((4 * NPAD,), jnp.float32),
        pltpu.VMEM((CHUNK,), jnp.float32),
        pltpu.VMEM((L,), jnp.float32),
        pltpu.VMEM((NTILES * 8,), jnp.float32),
        pltpu.VMEM((64 * L,), jnp.float32),
        pltpu.VMEM_SHARED((2 * NTILES * 8,), jnp.float32),
        pltpu.SemaphoreType.DMA,
    ],
)


@jax.jit
def kernel(boxes, scores):
    out = _nms_call(boxes.reshape(-1), scores)
    return out.reshape(64, L)[:TOPK, :5]


# speedup vs baseline: 1.0795x; 1.0795x over previous
import functools

import jax
import jax.numpy as jnp
from jax import lax
from jax.experimental import pallas as pl
from jax.experimental.pallas import tpu as pltpu
from jax.experimental.pallas import tpu_sc as plsc

N = 5000
TOPK = 50
IOU_THRESH = 0.5

L = 16
NTILES = 16
NPAD = 5120
CHUNK = NPAD // NTILES
NVREG = CHUNK // L
LAST = N - (NTILES - 1) * CHUNK
NEG = float("-inf")
BIG = 1 << 30


def _nms_body(bxh, sch, outh, bb, scl, pub, loc, outv, sh, sem):
    cid = lax.axis_index("c")
    tid = lax.axis_index("s")

    @pl.when(cid == 0)
    def _():
        base = tid * CHUNK
        lanes = jnp.arange(L, dtype=jnp.int32)

        cp = pltpu.async_copy(bxh, bb.at[pl.ds(0, 4 * N)], sem)

        @pl.when(tid < NTILES - 1)
        def _():
            pltpu.sync_copy(sch.at[pl.ds(base, CHUNK)], scl)

        @pl.when(tid == NTILES - 1)
        def _():
            pltpu.sync_copy(sch.at[pl.ds((NTILES - 1) * CHUNK, LAST)],
                            scl.at[pl.ds(0, LAST)])

        @pl.when(tid == 0)
        def _():
            zeros = jnp.zeros((L,), jnp.float32)
            for r in range(64):
                outv[pl.ds(r * L, L)] = zeros

        def publish(mloc, iloc, slot):
            pub[...] = jnp.where(
                lanes == 0, jnp.full((L,), mloc, jnp.float32),
                jnp.where(lanes == 1,
                          jnp.full((L,), iloc.astype(jnp.float32), jnp.float32),
                          jnp.zeros((L,), jnp.float32)))
            pltpu.sync_copy(pub.at[pl.ds(0, 8)],
                            sh.at[pl.ds(slot * (NTILES * 8) + tid * 8, 8)])

        def lane_reduce(bv, bi):
            mloc = jnp.max(bv, axis=0)
            iloc = jnp.min(jnp.where(bv == mloc, bi, BIG), axis=0)
            return mloc, iloc

        negs = jnp.full((L,), NEG, jnp.float32)
        svs = []
        for j in range(NVREG):
            v = scl[pl.ds(j * L, L)]
            svs.append(jnp.where(lanes + (base + j * L) < N, v, negs))

        bv0 = jnp.full((L,), NEG, jnp.float32)
        bi0 = lanes + base
        bv, bi = bv0, bi0
        for j in range(NVREG):
            ci = lanes + (base + j * L)
            upd = svs[j] > bv
            bv = jnp.where(upd, svs[j], bv)
            bi = jnp.where(upd, ci, bi)
        mloc, iloc = lane_reduce(bv, bi)
        publish(mloc, iloc, 0)
        cp.wait()
        plsc.subcore_barrier()

        def step(s, svs):
            svs = list(svs)
            slot = lax.rem(s, 2)
            nslot = lax.rem(s + 1, 2)

            pltpu.sync_copy(sh.at[pl.ds(slot * (NTILES * 8), NTILES * 8)], loc)
            vals = plsc.load_gather(loc, [lanes * 8])
            idxs = plsc.load_gather(loc, [lanes * 8 + 1]).astype(jnp.int32)
            m = jnp.max(vals, axis=0)
            wi = jnp.min(jnp.where(vals == m, idxs, BIG), axis=0)
            valid = m > NEG

            w4 = jnp.full((L,), wi * 4, jnp.int32)
            x1w = plsc.load_gather(bb, [w4])
            y1w = plsc.load_gather(bb, [w4 + 1])
            x2w = plsc.load_gather(bb, [w4 + 2])
            y2w = plsc.load_gather(bb, [w4 + 3])
            aw = (x2w - x1w) * (y2w - y1w)

            bv, bi = bv0, bi0
            for j in range(NVREG):
                off = base + j * L
                ci = lanes + off
                c4 = ci * 4
                x1 = plsc.load_gather(bb, [c4])
                y1 = plsc.load_gather(bb, [c4 + 1])
                x2 = plsc.load_gather(bb, [c4 + 2])
                y2 = plsc.load_gather(bb, [c4 + 3])
                a = (x2 - x1) * (y2 - y1)
                w = jnp.maximum(jnp.minimum(x2w, x2) - jnp.maximum(x1w, x1),
                                0.0)
                h = jnp.maximum(jnp.minimum(y2w, y2) - jnp.maximum(y1w, y1),
                                0.0)
                inter = w * h
                iou = inter / (aw + a - inter + jnp.float32(1e-8))
                kill = (iou >= IOU_THRESH) | (ci == wi)
                newv = jnp.where(kill, negs, svs[j])
                svs[j] = newv
                upd = newv > bv
                bv = jnp.where(upd, newv, bv)
                bi = jnp.where(upd, ci, bi)
            mloc, iloc = lane_reduce(bv, bi)
            publish(mloc, iloc, nslot)

            @pl.when(valid & (tid == 0))
            def _():
                msplat = jnp.full((L,), m, jnp.float32)
                zero = jnp.zeros((L,), jnp.float32)
                row = jnp.where(
                    lanes == 0, x1w,
                    jnp.where(lanes == 1, y1w,
                              jnp.where(lanes == 2, x2w,
                                        jnp.where(lanes == 3, y2w,
                                                  jnp.where(lanes == 4, msplat,
                                                            zero)))))
                outv[pl.ds(s * L, L)] = row

            plsc.subcore_barrier()
            return tuple(svs)

        lax.fori_loop(0, TOPK, step, tuple(svs))

        @pl.when(tid == 0)
        def _():
            pltpu.sync_copy(outv, outh)


_nms_call = pl.kernel(
    _nms_body,
    out_type=jax.ShapeDtypeStruct((64 * L,), jnp.float32),
    mesh=plsc.VectorSubcoreMesh(core_axis_name="c", subcore_axis_name="s"),
    compiler_params=pltpu.CompilerParams(needs_layout_passes=False),
    scratch_types=[
        pltpu.VMEM((4 * NPAD,), jnp.float32),
        pltpu.VMEM((CHUNK,), jnp.float32),
        pltpu.VMEM((L,), jnp.float32),
        pltpu.VMEM((NTILES * 8,), jnp.float32),
        pltpu.VMEM((64 * L,), jnp.float32),
        pltpu.VMEM_SHARED((2 * NTILES * 8,), jnp.float32),
        pltpu.SemaphoreType.DMA,
    ],
)


@jax.jit
def kernel(boxes, scores):
    out = _nms_call(boxes.reshape(-1), scores)
    return out.reshape(64, L)[:TOPK, :5]


# speedup vs baseline: 1.3494x; 1.2500x over previous
import functools

import jax
import jax.numpy as jnp
from jax import lax
from jax.experimental import pallas as pl
from jax.experimental.pallas import tpu as pltpu
from jax.experimental.pallas import tpu_sc as plsc

N = 5000
TOPK = 50
IOU_THRESH = 0.5

L = 16
NTILES = 16
NPAD = 5120
CHUNK = NPAD // NTILES
NVREG = CHUNK // L
LAST = N - (NTILES - 1) * CHUNK
NEG = float("-inf")
BIG = 1 << 30


def _nms_body(bxh, sch, outh, bb, scl, pub, loc, outv, sh, sem):
    cid = lax.axis_index("c")
    tid = lax.axis_index("s")

    @pl.when(cid == 0)
    def _():
        base = tid * CHUNK
        lanes = jnp.arange(L, dtype=jnp.int32)
        negs = jnp.full((L,), NEG, jnp.float32)

        cp = pltpu.async_copy(bxh, bb.at[pl.ds(0, 4 * N)], sem)

        @pl.when(tid < NTILES - 1)
        def _():
            pltpu.sync_copy(sch.at[pl.ds(base, CHUNK)], scl)

        @pl.when(tid == NTILES - 1)
        def _():
            pltpu.sync_copy(sch.at[pl.ds((NTILES - 1) * CHUNK, LAST)],
                            scl.at[pl.ds(0, LAST)])

        @pl.when(tid == 0)
        def _():
            zeros = jnp.zeros((L,), jnp.float32)
            for r in range(64):
                outv[pl.ds(r * L, L)] = zeros

        def top2_reduce(b1v, b1i, b2v, b2i):
            mv1 = jnp.max(b1v, axis=0)
            li1 = jnp.min(jnp.where(b1v == mv1, b1i, BIG), axis=0)
            winlane = (b1v == mv1) & (b1i == li1)
            c2v = jnp.where(winlane, b2v, b1v)
            c2i = jnp.where(winlane, b2i, b1i)
            mv2 = jnp.max(c2v, axis=0)
            li2 = jnp.min(jnp.where(c2v == mv2, c2i, BIG), axis=0)
            return mv1, li1, mv2, li2

        def publish(mv1, li1, mv2, li2, slot):
            li2c = jnp.minimum(li2, NPAD - 1)
            pub[...] = jnp.where(
                lanes == 0, jnp.full((L,), mv1, jnp.float32),
                jnp.where(lanes == 1,
                          jnp.full((L,), li1.astype(jnp.float32), jnp.float32),
                jnp.where(lanes == 2, jnp.full((L,), mv2, jnp.float32),
                jnp.where(lanes == 3,
                          jnp.full((L,), li2c.astype(jnp.float32), jnp.float32),
                          jnp.zeros((L,), jnp.float32)))))
            pltpu.sync_copy(pub.at[pl.ds(0, 8)],
                            sh.at[pl.ds(slot * (NTILES * 8) + tid * 8, 8)])

        def coords(idx_splat4):
            x1 = plsc.load_gather(bb, [idx_splat4])
            y1 = plsc.load_gather(bb, [idx_splat4 + 1])
            x2 = plsc.load_gather(bb, [idx_splat4 + 2])
            y2 = plsc.load_gather(bb, [idx_splat4 + 3])
            return x1, y1, x2, y2, (x2 - x1) * (y2 - y1)

        def iou_vs(ax1, ay1, ax2, ay2, aarea, x1, y1, x2, y2):
            a = (x2 - x1) * (y2 - y1)
            w = jnp.maximum(jnp.minimum(ax2, x2) - jnp.maximum(ax1, x1), 0.0)
            h = jnp.maximum(jnp.minimum(ay2, y2) - jnp.maximum(ay1, y1), 0.0)
            inter = w * h
            return inter / (aarea + a - inter + jnp.float32(1e-8))

        svs = []
        for j in range(NVREG):
            v = scl[pl.ds(j * L, L)]
            svs.append(jnp.where(lanes + (base + j * L) < N, v, negs))

        def local_top2(svs):
            b1v = b2v = negs
            b1i = b2i = lanes + base
            for j in range(NVREG):
                ci = lanes + (base + j * L)
                v = svs[j]
                u1 = v > b1v
                nb1v = jnp.where(u1, v, b1v)
                nb1i = jnp.where(u1, ci, b1i)
                dv = jnp.where(u1, b1v, v)
                di = jnp.where(u1, b1i, ci)
                u2 = dv > b2v
                b1v, b1i = nb1v, nb1i
                b2v = jnp.where(u2, dv, b2v)
                b2i = jnp.where(u2, di, b2i)
            return b1v, b1i, b2v, b2i

        mv1, li1, mv2, li2 = top2_reduce(*local_top2(svs))
        publish(mv1, li1, mv2, li2, 0)
        cp.wait()
        plsc.subcore_barrier()

        def round_body(carry):
            steps, rnd = carry[0], carry[1]
            svs = list(carry[2:])
            slot = lax.rem(rnd, 2)
            nslot = lax.rem(rnd + 1, 2)

            pltpu.sync_copy(sh.at[pl.ds(slot * (NTILES * 8), NTILES * 8)], loc)
            v1s = plsc.load_gather(loc, [lanes * 8])
            i1s = plsc.load_gather(loc, [lanes * 8 + 1]).astype(jnp.int32)
            v2s = plsc.load_gather(loc, [lanes * 8 + 2])
            i2s = plsc.load_gather(loc, [lanes * 8 + 3]).astype(jnp.int32)

            mA = jnp.max(v1s, axis=0)
            liA = jnp.min(jnp.where(v1s == mA, i1s, BIG), axis=0)
            validA = mA > NEG
            Ax1, Ay1, Ax2, Ay2, Aar = coords(jnp.full((L,), liA * 4, jnp.int32))

            liAs = jnp.full((L,), liA, jnp.int32)
            c1x1, c1y1, c1x2, c1y2, _ = coords(i1s * 4)
            c2x1, c2y1, c2x2, c2y2, _ = coords(i2s * 4)
            k1 = (iou_vs(Ax1, Ay1, Ax2, Ay2, Aar, c1x1, c1y1, c1x2, c1y2)
                  >= IOU_THRESH) | (i1s == liAs)
            k2 = (iou_vs(Ax1, Ay1, Ax2, Ay2, Aar, c2x1, c2y1, c2x2, c2y2)
                  >= IOU_THRESH) | (i2s == liAs)
            candv = jnp.where(~k1, v1s, jnp.where(~k2, v2s, negs))
            candi = jnp.where(~k1, i1s, i2s)
            unres = k1 & k2
            mB = jnp.max(candv, axis=0)
            bound = jnp.max(jnp.where(unres, v2s, negs), axis=0)
            liB = jnp.min(jnp.where(candv == mB, candi, BIG), axis=0)
            doB = mB > bound
            liBs = jnp.where(doB, liB, 0)
            Bx1, By1, Bx2, By2, Bar = coords(jnp.full((L,), liBs * 4,
                                                      jnp.int32))
            doBv = jnp.where(jnp.full((L,), doB, jnp.bool_),
                             jnp.full((L,), True, jnp.bool_),
                             jnp.full((L,), False, jnp.bool_))

            b1v = b2v = negs
            b1i = b2i = lanes + base
            for j in range(NVREG):
                ci = lanes + (base + j * L)
                c4 = ci * 4
                x1 = plsc.load_gather(bb, [c4])
                y1 = plsc.load_gather(bb, [c4 + 1])
                x2 = plsc.load_gather(bb, [c4 + 2])
                y2 = plsc.load_gather(bb, [c4 + 3])
                killA = (iou_vs(Ax1, Ay1, Ax2, Ay2, Aar, x1, y1, x2, y2)
                         >= IOU_THRESH) | (ci == liAs)
                killB = (iou_vs(Bx1, By1, Bx2, By2, Bar, x1, y1, x2, y2)
                         >= IOU_THRESH) | (ci == jnp.full((L,), liBs,
                                                          jnp.int32))
                kill = killA | (doBv & killB)
                v = jnp.where(kill, negs, svs[j])
                svs[j] = v
                u1 = v > b1v
                nb1v = jnp.where(u1, v, b1v)
                nb1i = jnp.where(u1, ci, b1i)
                dv = jnp.where(u1, b1v, v)
                di = jnp.where(u1, b1i, ci)
                u2 = dv > b2v
                b1v, b1i = nb1v, nb1i
                b2v = jnp.where(u2, dv, b2v)
                b2i = jnp.where(u2, di, b2i)

            mv1, li1, mv2, li2 = top2_reduce(b1v, b1i, b2v, b2i)
            publish(mv1, li1, mv2, li2, nslot)

            @pl.when(validA & (tid == 0))
            def _():
                zero = jnp.zeros((L,), jnp.float32)
                rowA = jnp.where(
                    lanes == 0, Ax1,
                    jnp.where(lanes == 1, Ay1,
                              jnp.where(lanes == 2, Ax2,
                                        jnp.where(lanes == 3, Ay2,
                                                  jnp.where(lanes == 4,
                                                            jnp.full((L,), mA,
                                                                     jnp.float32),
                                                            zero)))))
                outv[pl.ds(steps * L, L)] = rowA

                @pl.when(doB)
                def _():
                    rowB = jnp.where(
                        lanes == 0, Bx1,
                        jnp.where(lanes == 1, By1,
                                  jnp.where(lanes == 2, Bx2,
                                            jnp.where(lanes == 3, By2,
                                                      jnp.where(lanes == 4,
                                                                jnp.full((L,), mB,
                                                                         jnp.float32),
                                                                zero)))))
                    outv[pl.ds((steps + 1) * L, L)] = rowB

            plsc.subcore_barrier()
            nsteps = steps + jnp.where(doB, 2, 1).astype(jnp.int32)
            return tuple([nsteps, rnd + 1] + svs)

        def round_cond(carry):
            return carry[0] < TOPK

        init = tuple([jnp.int32(0), jnp.int32(0)] + svs)
        lax.while_loop(round_cond, round_body, init)

        @pl.when(tid == 0)
        def _():
            pltpu.sync_copy(outv, outh)


_nms_call = pl.kernel(
    _nms_body,
    out_type=jax.ShapeDtypeStruct((64 * L,), jnp.float32),
    mesh=plsc.VectorSubcoreMesh(core_axis_name="c", subcore_axis_name="s"),
    compiler_params=pltpu.CompilerParams(needs_layout_passes=False),
    scratch_types=[
        pltpu.VMEM((4 * NPAD,), jnp.float32),
        pltpu.VMEM((CHUNK,), jnp.float32),
        pltpu.VMEM((L,), jnp.float32),
        pltpu.VMEM((NTILES * 8,), jnp.float32),
        pltpu.VMEM((64 * L,), jnp.float32),
        pltpu.VMEM_SHARED((2 * NTILES * 8,), jnp.float32),
        pltpu.SemaphoreType.DMA,
    ],
)


@jax.jit
def kernel(boxes, scores):
    out = _nms_call(boxes.reshape(-1), scores)
    return out.reshape(64, L)[:TOPK, :5]
